# Initial kernel scaffold; baseline (speedup 1.0000x reference)
import jax
import jax.numpy as jnp
from jax.experimental import pallas as pl


def kernel(x, edge_index, dropout, W1, root1, bias1, W2, root2, bias2):
    raise NotImplementedError("write your pallas kernel here")


# speedup vs baseline: 9.7664x; 9.7664x over previous
import functools

import jax
import jax.numpy as jnp
from jax import lax
from jax.experimental import pallas as pl
from jax.experimental.pallas import tpu as pltpu
from jax.experimental.pallas import tpu_sc as plsc

N_NODES = 10000
D_IN = 128
D_HID = 16

U = 0.1

NC = 2
NS = 16
NW = NC * NS
CHUNK = 128
ACC_ROWS = 10240
ROWS_PER_TILE = ACC_ROWS // NS
DUMMY_DST = N_NODES

BLK = 1000


def _num_chunks(n_edges):
    k = -(-n_edges // (NW * CHUNK))
    return -(-k // 8) * 8


def _tc1_body(x_ref, w1_ref, root1_ref, y_ref, xr_ref):
    wc = (1.0 - U) * w1_ref[0] + U * w1_ref[1]
    x = x_ref[...]
    y_ref[...] = jnp.dot(x, wc, preferred_element_type=jnp.float32)
    xr_ref[...] = jnp.dot(x, root1_ref[...], preferred_element_type=jnp.float32)


def _tc_stage1(x, W1, root1):
    n = x.shape[0]
    return pl.pallas_call(
        _tc1_body,
        grid=(n // BLK,),
        in_specs=[
            pl.BlockSpec((BLK, D_IN), lambda i: (i, 0)),
            pl.BlockSpec((2, D_IN, D_HID), lambda i: (0, 0, 0)),
            pl.BlockSpec((D_IN, D_HID), lambda i: (0, 0)),
        ],
        out_specs=[
            pl.BlockSpec((BLK, D_HID), lambda i: (i, 0)),
            pl.BlockSpec((BLK, D_HID), lambda i: (i, 0)),
        ],
        out_shape=[
            jax.ShapeDtypeStruct((n, D_HID), jnp.float32),
            jax.ShapeDtypeStruct((n, D_HID), jnp.float32),
        ],
    )(x, W1, root1)


def _make_sc_pass(n_chunks, with_deg):
    mesh = plsc.VectorSubcoreMesh(core_axis_name="c", subcore_axis_name="s")

    out_type = [jax.ShapeDtypeStruct((NC, ACC_ROWS, D_HID), jnp.float32)]
    scratch = [
        pltpu.VMEM((n_chunks, CHUNK), jnp.int32),
        pltpu.VMEM((n_chunks, CHUNK), jnp.int32),
        pltpu.VMEM((CHUNK, D_HID), jnp.float32),
        pltpu.VMEM_SHARED((ACC_ROWS, D_HID), jnp.float32),
        pltpu.SemaphoreType.DMA,
    ]
    if with_deg:
        out_type.append(jax.ShapeDtypeStruct((NC, ACC_ROWS, D_HID), jnp.float32))
        scratch.insert(3, pltpu.VMEM((CHUNK, D_HID), jnp.float32))
        scratch.insert(5, pltpu.VMEM_SHARED((ACC_ROWS, D_HID), jnp.float32))

    @functools.partial(
        pl.kernel, mesh=mesh, out_type=out_type, scratch_types=scratch,
        compiler_params=pltpu.CompilerParams(use_tc_tiling_on_sc=False),
    )
    def sc_pass(*refs):
        if with_deg:
            (src2d, dst2d, y_hbm, zeros_hbm, ones_hbm,
             outy, outd,
             sidx, didx, rows, onesb, accy, accd, sem) = refs
        else:
            (src2d, dst2d, y_hbm, zeros_hbm,
             outy,
             sidx, didx, rows, accy, sem) = refs

        cid = lax.axis_index("c")
        sid = lax.axis_index("s")
        wid = cid * NS + sid

        tile_rows = pl.ds(sid * ROWS_PER_TILE, ROWS_PER_TILE)
        pltpu.sync_copy(zeros_hbm, accy.at[tile_rows])
        if with_deg:
            pltpu.sync_copy(zeros_hbm, accd.at[tile_rows])
            pltpu.sync_copy(ones_hbm, onesb)
        plsc.subcore_barrier()

        erow = pl.ds(wid * n_chunks, n_chunks)
        pltpu.sync_copy(src2d.at[erow], sidx)
        pltpu.sync_copy(dst2d.at[erow], didx)

        def step(j, carry):
            pltpu.async_copy(y_hbm.at[sidx.at[j]], rows, sem).wait()
            pltpu.sync_copy(rows, accy.at[didx.at[j]], add=True)
            if with_deg:
                pltpu.sync_copy(onesb, accd.at[didx.at[j]], add=True)
            return carry

        lax.fori_loop(0, n_chunks, step, 0)
        plsc.subcore_barrier()

        pltpu.sync_copy(accy.at[tile_rows], outy.at[cid].at[tile_rows])
        if with_deg:
            pltpu.sync_copy(accd.at[tile_rows], outd.at[cid].at[tile_rows])

    return sc_pass


def _tc2_body(accy_ref, accd_ref, xr_ref, b1_ref, w2_ref, root2_ref,
              y2_ref, hr_ref):
    agg = accy_ref[0] + accy_ref[1]
    deg = accd_ref[0, :, 0:1] + accd_ref[1, :, 0:1]
    h = agg / jnp.maximum(deg, 1.0) + xr_ref[...] + b1_ref[...]
    h = jnp.where(h > 0.0, h, jnp.exp(jnp.minimum(h, 0.0)) - 1.0)
    wc = (1.0 - U) * w2_ref[0] + U * w2_ref[1]
    y2_ref[...] = jnp.dot(h, wc, preferred_element_type=jnp.float32)
    hr_ref[...] = jnp.dot(h, root2_ref[...], preferred_element_type=jnp.float32)


def _tc_stage2(accY, accD, XR, bias1, W2, root2):
    n = XR.shape[0]
    return pl.pallas_call(
        _tc2_body,
        grid=(n // BLK,),
        in_specs=[
            pl.BlockSpec((NC, BLK, D_HID), lambda i: (0, i, 0)),
            pl.BlockSpec((NC, BLK, D_HID), lambda i: (0, i, 0)),
            pl.BlockSpec((BLK, D_HID), lambda i: (i, 0)),
            pl.BlockSpec((1, D_HID), lambda i: (0, 0)),
            pl.BlockSpec((2, D_HID, D_HID), lambda i: (0, 0, 0)),
            pl.BlockSpec((D_HID, D_HID), lambda i: (0, 0)),
        ],
        out_specs=[
            pl.BlockSpec((BLK, D_HID), lambda i: (i, 0)),
            pl.BlockSpec((BLK, D_HID), lambda i: (i, 0)),
        ],
        out_shape=[
            jax.ShapeDtypeStruct((n, D_HID), jnp.float32),
            jax.ShapeDtypeStruct((n, D_HID), jnp.float32),
        ],
    )(accY, accD, XR, bias1, W2, root2)


def _tc3_body(accy_ref, accd_ref, hr_ref, b2_ref, out_ref):
    agg = accy_ref[0] + accy_ref[1]
    deg = accd_ref[0, :, 0:1] + accd_ref[1, :, 0:1]
    o = agg / jnp.maximum(deg, 1.0) + hr_ref[...] + b2_ref[...]
    m = jnp.max(o, axis=1, keepdims=True)
    s = o - m
    lse = jnp.log(jnp.sum(jnp.exp(s), axis=1, keepdims=True))
    out_ref[...] = s - lse


def _tc_stage3(accY2, accD, HR, bias2):
    n = HR.shape[0]
    return pl.pallas_call(
        _tc3_body,
        grid=(n // BLK,),
        in_specs=[
            pl.BlockSpec((NC, BLK, D_HID), lambda i: (0, i, 0)),
            pl.BlockSpec((NC, BLK, D_HID), lambda i: (0, i, 0)),
            pl.BlockSpec((BLK, D_HID), lambda i: (i, 0)),
            pl.BlockSpec((1, D_HID), lambda i: (0, 0)),
        ],
        out_specs=pl.BlockSpec((BLK, D_HID), lambda i: (i, 0)),
        out_shape=jax.ShapeDtypeStruct((n, D_HID), jnp.float32),
    )(accY2, accD, HR, bias2)


def kernel(x, edge_index, dropout, W1, root1, bias1, W2, root2, bias2):
    n = x.shape[0]
    e = edge_index.shape[1]
    n_chunks = _num_chunks(e)
    e_pad = NW * n_chunks * CHUNK

    src = edge_index[0].astype(jnp.int32)
    dst = edge_index[1].astype(jnp.int32)
    src = jnp.concatenate([src, jnp.zeros((e_pad - e,), jnp.int32)])
    dst = jnp.concatenate(
        [dst, jnp.full((e_pad - e,), DUMMY_DST, jnp.int32)])
    src2d = src.reshape(e_pad // CHUNK, CHUNK)
    dst2d = dst.reshape(e_pad // CHUNK, CHUNK)

    zeros_t = jnp.zeros((ROWS_PER_TILE, D_HID), jnp.float32)
    ones_t = jnp.ones((CHUNK, D_HID), jnp.float32)

    Y1, XR = _tc_stage1(x, W1, root1)

    sc_a = _make_sc_pass(n_chunks, with_deg=True)
    accY, accD = sc_a(src2d, dst2d, Y1, zeros_t, ones_t)

    Y2, HR = _tc_stage2(accY[:, :n], accD[:, :n], XR,
                        bias1.reshape(1, D_HID), W2, root2)

    sc_b = _make_sc_pass(n_chunks, with_deg=False)
    res = sc_b(src2d, dst2d, Y2, zeros_t)
    accY2 = res[0] if isinstance(res, (list, tuple)) else res

    return _tc_stage3(accY2[:, :n], accD[:, :n], HR,
                      bias2.reshape(1, D_HID))


# speedup vs baseline: 13.8904x; 1.4223x over previous
import functools

import jax
import jax.numpy as jnp
from jax import lax
from jax.experimental import pallas as pl
from jax.experimental.pallas import tpu as pltpu
from jax.experimental.pallas import tpu_sc as plsc

N_NODES = 10000
D_IN = 128
D_HID = 16

U = 0.1

NC = 2
NS = 16
NW = NC * NS
CHUNK = 128
ACC_ROWS = 10240
ROWS_PER_TILE = ACC_ROWS // NS
DUMMY_DST = N_NODES
DEG_W = 16

BLK = 1000


def _num_chunks(n_edges):
    k = -(-n_edges // (NW * CHUNK))
    return -(-k // 8) * 8


def _tc1_body(x_ref, w1_ref, root1_ref, y_ref, xr_ref):
    wc = (1.0 - U) * w1_ref[0] + U * w1_ref[1]
    x = x_ref[...]
    y_ref[...] = jnp.dot(x, wc, preferred_element_type=jnp.float32)
    xr_ref[...] = jnp.dot(x, root1_ref[...], preferred_element_type=jnp.float32)


def _tc_stage1(x, W1, root1):
    n = x.shape[0]
    return pl.pallas_call(
        _tc1_body,
        grid=(n // BLK,),
        in_specs=[
            pl.BlockSpec((BLK, D_IN), lambda i: (i, 0)),
            pl.BlockSpec((2, D_IN, D_HID), lambda i: (0, 0, 0)),
            pl.BlockSpec((D_IN, D_HID), lambda i: (0, 0)),
        ],
        out_specs=[
            pl.BlockSpec((BLK, D_HID), lambda i: (i, 0)),
            pl.BlockSpec((BLK, D_HID), lambda i: (i, 0)),
        ],
        out_shape=[
            jax.ShapeDtypeStruct((n, D_HID), jnp.float32),
            jax.ShapeDtypeStruct((n, D_HID), jnp.float32),
        ],
    )(x, W1, root1)


def _make_sc_pass(n_chunks, with_deg):
    mesh = plsc.VectorSubcoreMesh(core_axis_name="c", subcore_axis_name="s")

    out_type = [jax.ShapeDtypeStruct((NC, ACC_ROWS, D_HID), jnp.float32)]
    scratch = [
        pltpu.VMEM((n_chunks, CHUNK), jnp.int32),
        pltpu.VMEM((n_chunks, CHUNK), jnp.int32),
        pltpu.VMEM((CHUNK, D_HID), jnp.float32),
        pltpu.VMEM((CHUNK, D_HID), jnp.float32),
        pltpu.VMEM_SHARED((ACC_ROWS, D_HID), jnp.float32),
        pltpu.SemaphoreType.DMA,
        pltpu.SemaphoreType.DMA,
    ]
    if with_deg:
        out_type.append(jax.ShapeDtypeStruct((NC, ACC_ROWS, DEG_W), jnp.float32))
        scratch.insert(4, pltpu.VMEM((CHUNK, DEG_W), jnp.float32))
        scratch.insert(6, pltpu.VMEM_SHARED((ACC_ROWS, DEG_W), jnp.float32))

    @functools.partial(
        pl.kernel, mesh=mesh, out_type=out_type, scratch_types=scratch,
        compiler_params=pltpu.CompilerParams(use_tc_tiling_on_sc=False),
    )
    def sc_pass(*refs):
        if with_deg:
            (src2d, dst2d, y_hbm, zeros_hbm, zeros1_hbm, ones_hbm,
             outy, outd,
             sidx, didx, bufa, bufb, onesb, accy, accd, sema, semb) = refs
        else:
            (src2d, dst2d, y_hbm, zeros_hbm,
             outy,
             sidx, didx, bufa, bufb, accy, sema, semb) = refs

        cid = lax.axis_index("c")
        sid = lax.axis_index("s")
        wid = cid * NS + sid

        tile_rows = pl.ds(sid * ROWS_PER_TILE, ROWS_PER_TILE)
        pltpu.sync_copy(zeros_hbm, accy.at[tile_rows])
        if with_deg:
            pltpu.sync_copy(zeros1_hbm, accd.at[tile_rows])
            pltpu.sync_copy(ones_hbm, onesb)
        plsc.subcore_barrier()

        erow = pl.ds(wid * n_chunks, n_chunks)
        pltpu.sync_copy(src2d.at[erow], sidx)
        pltpu.sync_copy(dst2d.at[erow], didx)

        pltpu.async_copy(y_hbm.at[sidx.at[0]], bufa, sema)

        def pair(i, carry):
            j0 = 2 * i
            j1 = j0 + 1
            j2 = jnp.minimum(j0 + 2, n_chunks - 1)
            pltpu.async_copy(y_hbm.at[sidx.at[j1]], bufb, semb)
            pltpu.make_async_copy(y_hbm.at[sidx.at[j0]], bufa, sema).wait()
            pltpu.sync_copy(bufa, accy.at[didx.at[j0]], add=True)
            if with_deg:
                pltpu.sync_copy(onesb, accd.at[didx.at[j0]], add=True)
            pltpu.async_copy(y_hbm.at[sidx.at[j2]], bufa, sema)
            pltpu.make_async_copy(y_hbm.at[sidx.at[j1]], bufb, semb).wait()
            pltpu.sync_copy(bufb, accy.at[didx.at[j1]], add=True)
            if with_deg:
                pltpu.sync_copy(onesb, accd.at[didx.at[j1]], add=True)
            return carry

        lax.fori_loop(0, n_chunks // 2, pair, 0)
        pltpu.make_async_copy(
            y_hbm.at[sidx.at[n_chunks - 1]], bufa, sema).wait()
        plsc.subcore_barrier()

        pltpu.sync_copy(accy.at[tile_rows], outy.at[cid].at[tile_rows])
        if with_deg:
            pltpu.sync_copy(accd.at[tile_rows], outd.at[cid].at[tile_rows])

    return sc_pass


def _tc2_body(accy_ref, accd_ref, xr_ref, b1_ref, w2_ref, root2_ref,
              y2_ref, hr_ref):
    agg = accy_ref[0] + accy_ref[1]
    deg = accd_ref[0, :, 0:1] + accd_ref[1, :, 0:1]
    h = agg / jnp.maximum(deg, 1.0) + xr_ref[...] + b1_ref[...]
    h = jnp.where(h > 0.0, h, jnp.exp(jnp.minimum(h, 0.0)) - 1.0)
    wc = (1.0 - U) * w2_ref[0] + U * w2_ref[1]
    y2_ref[...] = jnp.dot(h, wc, preferred_element_type=jnp.float32)
    hr_ref[...] = jnp.dot(h, root2_ref[...], preferred_element_type=jnp.float32)


def _tc_stage2(accY, accD, XR, bias1, W2, root2):
    n = XR.shape[0]
    return pl.pallas_call(
        _tc2_body,
        grid=(n // BLK,),
        in_specs=[
            pl.BlockSpec((NC, BLK, D_HID), lambda i: (0, i, 0)),
            pl.BlockSpec((NC, BLK, DEG_W), lambda i: (0, i, 0)),
            pl.BlockSpec((BLK, D_HID), lambda i: (i, 0)),
            pl.BlockSpec((1, D_HID), lambda i: (0, 0)),
            pl.BlockSpec((2, D_HID, D_HID), lambda i: (0, 0, 0)),
            pl.BlockSpec((D_HID, D_HID), lambda i: (0, 0)),
        ],
        out_specs=[
            pl.BlockSpec((BLK, D_HID), lambda i: (i, 0)),
            pl.BlockSpec((BLK, D_HID), lambda i: (i, 0)),
        ],
        out_shape=[
            jax.ShapeDtypeStruct((n, D_HID), jnp.float32),
            jax.ShapeDtypeStruct((n, D_HID), jnp.float32),
        ],
    )(accY, accD, XR, bias1, W2, root2)


def _tc3_body(accy_ref, accd_ref, hr_ref, b2_ref, out_ref):
    agg = accy_ref[0] + accy_ref[1]
    deg = accd_ref[0, :, 0:1] + accd_ref[1, :, 0:1]
    o = agg / jnp.maximum(deg, 1.0) + hr_ref[...] + b2_ref[...]
    m = jnp.max(o, axis=1, keepdims=True)
    s = o - m
    lse = jnp.log(jnp.sum(jnp.exp(s), axis=1, keepdims=True))
    out_ref[...] = s - lse


def _tc_stage3(accY2, accD, HR, bias2):
    n = HR.shape[0]
    return pl.pallas_call(
        _tc3_body,
        grid=(n // BLK,),
        in_specs=[
            pl.BlockSpec((NC, BLK, D_HID), lambda i: (0, i, 0)),
            pl.BlockSpec((NC, BLK, DEG_W), lambda i: (0, i, 0)),
            pl.BlockSpec((BLK, D_HID), lambda i: (i, 0)),
            pl.BlockSpec((1, D_HID), lambda i: (0, 0)),
        ],
        out_specs=pl.BlockSpec((BLK, D_HID), lambda i: (i, 0)),
        out_shape=jax.ShapeDtypeStruct((n, D_HID), jnp.float32),
    )(accY2, accD, HR, bias2)


def kernel(x, edge_index, dropout, W1, root1, bias1, W2, root2, bias2):
    n = x.shape[0]
    e = edge_index.shape[1]
    n_chunks = _num_chunks(e)
    e_pad = NW * n_chunks * CHUNK

    src = edge_index[0].astype(jnp.int32)
    dst = edge_index[1].astype(jnp.int32)
    src = jnp.concatenate([src, jnp.zeros((e_pad - e,), jnp.int32)])
    dst = jnp.concatenate(
        [dst, jnp.full((e_pad - e,), DUMMY_DST, jnp.int32)])
    src2d = src.reshape(e_pad // CHUNK, CHUNK)
    dst2d = dst.reshape(e_pad // CHUNK, CHUNK)

    zeros_t = jnp.zeros((ROWS_PER_TILE, D_HID), jnp.float32)
    zeros1_t = jnp.zeros((ROWS_PER_TILE, DEG_W), jnp.float32)
    ones_t = jnp.ones((CHUNK, DEG_W), jnp.float32)

    Y1, XR = _tc_stage1(x, W1, root1)

    sc_a = _make_sc_pass(n_chunks, with_deg=True)
    accY, accD = sc_a(src2d, dst2d, Y1, zeros_t, zeros1_t, ones_t)

    Y2, HR = _tc_stage2(accY, accD, XR,
                        bias1.reshape(1, D_HID), W2, root2)

    sc_b = _make_sc_pass(n_chunks, with_deg=False)
    res = sc_b(src2d, dst2d, Y2, zeros_t)
    accY2 = res[0] if isinstance(res, (list, tuple)) else res

    return _tc_stage3(accY2, accD, HR,
                      bias2.reshape(1, D_HID))


# speedup vs baseline: 14.5010x; 1.0440x over previous
import functools

import jax
import jax.numpy as jnp
from jax import lax
from jax.experimental import pallas as pl
from jax.experimental.pallas import tpu as pltpu
from jax.experimental.pallas import tpu_sc as plsc

N_NODES = 10000
D_IN = 128
D_HID = 16

U = 0.1

NC = 2
NS = 16
NW = NC * NS
CHUNK = 128
ACC_ROWS = 10240
ROWS_PER_TILE = ACC_ROWS // NS
DUMMY_DST = N_NODES
DEG_W = 8

BLK = 1000


def _num_chunks(n_edges):
    k = -(-n_edges // (NW * CHUNK))
    return -(-k // 8) * 8


def _tc1_body(x_ref, w1_ref, root1_ref, y_ref, xr_ref):
    wc = (1.0 - U) * w1_ref[0] + U * w1_ref[1]
    x = x_ref[...]
    y_ref[...] = jnp.dot(x, wc, preferred_element_type=jnp.float32)
    xr_ref[...] = jnp.dot(x, root1_ref[...], preferred_element_type=jnp.float32)


def _tc_stage1(x, W1, root1):
    n = x.shape[0]
    return pl.pallas_call(
        _tc1_body,
        grid=(n // BLK,),
        in_specs=[
            pl.BlockSpec((BLK, D_IN), lambda i: (i, 0)),
            pl.BlockSpec((2, D_IN, D_HID), lambda i: (0, 0, 0)),
            pl.BlockSpec((D_IN, D_HID), lambda i: (0, 0)),
        ],
        out_specs=[
            pl.BlockSpec((BLK, D_HID), lambda i: (i, 0)),
            pl.BlockSpec((BLK, D_HID), lambda i: (i, 0)),
        ],
        out_shape=[
            jax.ShapeDtypeStruct((n, D_HID), jnp.float32),
            jax.ShapeDtypeStruct((n, D_HID), jnp.float32),
        ],
    )(x, W1, root1)


def _make_sc_pass(n_chunks, with_deg):
    mesh = plsc.VectorSubcoreMesh(core_axis_name="c", subcore_axis_name="s")

    out_type = [jax.ShapeDtypeStruct((NC, ACC_ROWS, D_HID), jnp.float32)]
    scratch = [
        pltpu.VMEM((n_chunks, CHUNK), jnp.int32),
        pltpu.VMEM((n_chunks, CHUNK), jnp.int32),
        pltpu.VMEM((CHUNK, D_HID), jnp.float32),
        pltpu.VMEM((CHUNK, D_HID), jnp.float32),
        pltpu.VMEM_SHARED((ACC_ROWS, D_HID), jnp.float32),
        pltpu.SemaphoreType.DMA,
        pltpu.SemaphoreType.DMA,
    ]
    if with_deg:
        out_type.append(jax.ShapeDtypeStruct((NC, ACC_ROWS, DEG_W), jnp.float32))
        scratch.insert(4, pltpu.VMEM((CHUNK, DEG_W), jnp.float32))
        scratch.insert(6, pltpu.VMEM_SHARED((ACC_ROWS, DEG_W), jnp.float32))

    @functools.partial(
        pl.kernel, mesh=mesh, out_type=out_type, scratch_types=scratch,
        compiler_params=pltpu.CompilerParams(use_tc_tiling_on_sc=False),
    )
    def sc_pass(*refs):
        if with_deg:
            (src2d, dst2d, y_hbm, zeros_hbm, zeros1_hbm, ones_hbm,
             outy, outd,
             sidx, didx, bufa, bufb, onesb, accy, accd, sema, semb) = refs
        else:
            (src2d, dst2d, y_hbm, zeros_hbm,
             outy,
             sidx, didx, bufa, bufb, accy, sema, semb) = refs

        cid = lax.axis_index("c")
        sid = lax.axis_index("s")
        wid = cid * NS + sid

        tile_rows = pl.ds(sid * ROWS_PER_TILE, ROWS_PER_TILE)
        pltpu.sync_copy(zeros_hbm, accy.at[tile_rows])
        if with_deg:
            pltpu.sync_copy(zeros1_hbm, accd.at[tile_rows])
            pltpu.sync_copy(ones_hbm, onesb)
        plsc.subcore_barrier()

        erow = pl.ds(wid * n_chunks, n_chunks)
        pltpu.sync_copy(src2d.at[erow], sidx)
        pltpu.sync_copy(dst2d.at[erow], didx)

        pltpu.async_copy(y_hbm.at[sidx.at[0]], bufa, sema)

        def pair(i, carry):
            j0 = 2 * i
            j1 = j0 + 1
            j2 = jnp.minimum(j0 + 2, n_chunks - 1)
            pltpu.async_copy(y_hbm.at[sidx.at[j1]], bufb, semb)
            pltpu.make_async_copy(y_hbm.at[sidx.at[j0]], bufa, sema).wait()
            pltpu.sync_copy(bufa, accy.at[didx.at[j0]], add=True)
            if with_deg:
                pltpu.sync_copy(onesb, accd.at[didx.at[j0]], add=True)
            pltpu.async_copy(y_hbm.at[sidx.at[j2]], bufa, sema)
            pltpu.make_async_copy(y_hbm.at[sidx.at[j1]], bufb, semb).wait()
            pltpu.sync_copy(bufb, accy.at[didx.at[j1]], add=True)
            if with_deg:
                pltpu.sync_copy(onesb, accd.at[didx.at[j1]], add=True)
            return carry

        lax.fori_loop(0, n_chunks // 2, pair, 0)
        pltpu.make_async_copy(
            y_hbm.at[sidx.at[n_chunks - 1]], bufa, sema).wait()
        plsc.subcore_barrier()

        pltpu.sync_copy(accy.at[tile_rows], outy.at[cid].at[tile_rows])
        if with_deg:
            pltpu.sync_copy(accd.at[tile_rows], outd.at[cid].at[tile_rows])

    return sc_pass


def _tc2_body(accy_ref, accd_ref, xr_ref, b1_ref, w2_ref, root2_ref,
              y2_ref, hr_ref):
    agg = accy_ref[0] + accy_ref[1]
    deg = accd_ref[0, :, 0:1] + accd_ref[1, :, 0:1]
    h = agg / jnp.maximum(deg, 1.0) + xr_ref[...] + b1_ref[...]
    h = jnp.where(h > 0.0, h, jnp.exp(jnp.minimum(h, 0.0)) - 1.0)
    wc = (1.0 - U) * w2_ref[0] + U * w2_ref[1]
    y2_ref[...] = jnp.dot(h, wc, preferred_element_type=jnp.float32)
    hr_ref[...] = jnp.dot(h, root2_ref[...], preferred_element_type=jnp.float32)


def _tc_stage2(accY, accD, XR, bias1, W2, root2):
    n = XR.shape[0]
    return pl.pallas_call(
        _tc2_body,
        grid=(n // BLK,),
        in_specs=[
            pl.BlockSpec((NC, BLK, D_HID), lambda i: (0, i, 0)),
            pl.BlockSpec((NC, BLK, DEG_W), lambda i: (0, i, 0)),
            pl.BlockSpec((BLK, D_HID), lambda i: (i, 0)),
            pl.BlockSpec((1, D_HID), lambda i: (0, 0)),
            pl.BlockSpec((2, D_HID, D_HID), lambda i: (0, 0, 0)),
            pl.BlockSpec((D_HID, D_HID), lambda i: (0, 0)),
        ],
        out_specs=[
            pl.BlockSpec((BLK, D_HID), lambda i: (i, 0)),
            pl.BlockSpec((BLK, D_HID), lambda i: (i, 0)),
        ],
        out_shape=[
            jax.ShapeDtypeStruct((n, D_HID), jnp.float32),
            jax.ShapeDtypeStruct((n, D_HID), jnp.float32),
        ],
    )(accY, accD, XR, bias1, W2, root2)


def _tc3_body(accy_ref, accd_ref, hr_ref, b2_ref, out_ref):
    agg = accy_ref[0] + accy_ref[1]
    deg = accd_ref[0, :, 0:1] + accd_ref[1, :, 0:1]
    o = agg / jnp.maximum(deg, 1.0) + hr_ref[...] + b2_ref[...]
    m = jnp.max(o, axis=1, keepdims=True)
    s = o - m
    lse = jnp.log(jnp.sum(jnp.exp(s), axis=1, keepdims=True))
    out_ref[...] = s - lse


def _tc_stage3(accY2, accD, HR, bias2):
    n = HR.shape[0]
    return pl.pallas_call(
        _tc3_body,
        grid=(n // BLK,),
        in_specs=[
            pl.BlockSpec((NC, BLK, D_HID), lambda i: (0, i, 0)),
            pl.BlockSpec((NC, BLK, DEG_W), lambda i: (0, i, 0)),
            pl.BlockSpec((BLK, D_HID), lambda i: (i, 0)),
            pl.BlockSpec((1, D_HID), lambda i: (0, 0)),
        ],
        out_specs=pl.BlockSpec((BLK, D_HID), lambda i: (i, 0)),
        out_shape=jax.ShapeDtypeStruct((n, D_HID), jnp.float32),
    )(accY2, accD, HR, bias2)


def kernel(x, edge_index, dropout, W1, root1, bias1, W2, root2, bias2):
    n = x.shape[0]
    e = edge_index.shape[1]
    n_chunks = _num_chunks(e)
    e_pad = NW * n_chunks * CHUNK

    src = edge_index[0].astype(jnp.int32)
    dst = edge_index[1].astype(jnp.int32)
    src = jnp.concatenate([src, jnp.zeros((e_pad - e,), jnp.int32)])
    dst = jnp.concatenate(
        [dst, jnp.full((e_pad - e,), DUMMY_DST, jnp.int32)])
    src2d = src.reshape(e_pad // CHUNK, CHUNK)
    dst2d = dst.reshape(e_pad // CHUNK, CHUNK)

    zeros_t = jnp.zeros((ROWS_PER_TILE, D_HID), jnp.float32)
    zeros1_t = jnp.zeros((ROWS_PER_TILE, DEG_W), jnp.float32)
    ones_t = jnp.ones((CHUNK, DEG_W), jnp.float32)

    Y1, XR = _tc_stage1(x, W1, root1)

    sc_a = _make_sc_pass(n_chunks, with_deg=True)
    accY, accD = sc_a(src2d, dst2d, Y1, zeros_t, zeros1_t, ones_t)

    Y2, HR = _tc_stage2(accY, accD, XR,
                        bias1.reshape(1, D_HID), W2, root2)

    sc_b = _make_sc_pass(n_chunks, with_deg=False)
    res = sc_b(src2d, dst2d, Y2, zeros_t)
    accY2 = res[0] if isinstance(res, (list, tuple)) else res

    return _tc_stage3(accY2, accD, HR,
                      bias2.reshape(1, D_HID))


# speedup vs baseline: 14.5257x; 1.0017x over previous
import functools

import jax
import jax.numpy as jnp
from jax import lax
from jax.experimental import pallas as pl
from jax.experimental.pallas import tpu as pltpu
from jax.experimental.pallas import tpu_sc as plsc

N_NODES = 10000
D_IN = 128
D_HID = 16

U = 0.1

NC = 2
NS = 16
NW = NC * NS
CHUNK = 128
ACC_ROWS = 10240
ROWS_PER_TILE = ACC_ROWS // NS
DUMMY_DST = N_NODES
DEG_W = 8

BLK = 1000


def _num_chunks(n_edges):
    k = -(-n_edges // (NW * CHUNK))
    return -(-k // 8) * 8


def _tc1_body(x_ref, w1_ref, root1_ref, y_ref, xr_ref):
    wc = (1.0 - U) * w1_ref[0] + U * w1_ref[1]
    x = x_ref[...]
    y_ref[...] = jnp.dot(x, wc, preferred_element_type=jnp.float32)
    xr_ref[...] = jnp.dot(x, root1_ref[...], preferred_element_type=jnp.float32)


def _tc_stage1(x, W1, root1):
    n = x.shape[0]
    return pl.pallas_call(
        _tc1_body,
        grid=(n // BLK,),
        in_specs=[
            pl.BlockSpec((BLK, D_IN), lambda i: (i, 0)),
            pl.BlockSpec((2, D_IN, D_HID), lambda i: (0, 0, 0)),
            pl.BlockSpec((D_IN, D_HID), lambda i: (0, 0)),
        ],
        out_specs=[
            pl.BlockSpec((BLK, D_HID), lambda i: (i, 0)),
            pl.BlockSpec((BLK, D_HID), lambda i: (i, 0)),
        ],
        out_shape=[
            jax.ShapeDtypeStruct((n, D_HID), jnp.float32),
            jax.ShapeDtypeStruct((n, D_HID), jnp.float32),
        ],
    )(x, W1, root1)


def _make_sc_pass(n_chunks, with_deg):
    mesh = plsc.VectorSubcoreMesh(core_axis_name="c", subcore_axis_name="s")

    out_type = [jax.ShapeDtypeStruct((NC, ACC_ROWS, D_HID), jnp.float32)]
    scratch = [
        pltpu.VMEM((n_chunks, CHUNK), jnp.int32),
        pltpu.VMEM((n_chunks, CHUNK), jnp.int32),
        pltpu.VMEM((CHUNK, D_HID), jnp.float32),
        pltpu.VMEM((CHUNK, D_HID), jnp.float32),
        pltpu.VMEM_SHARED((ACC_ROWS, D_HID), jnp.float32),
        pltpu.SemaphoreType.DMA,
        pltpu.SemaphoreType.DMA,
    ]
    if with_deg:
        out_type.append(jax.ShapeDtypeStruct((NC, ACC_ROWS, DEG_W), jnp.float32))
        scratch.insert(4, pltpu.VMEM((CHUNK, DEG_W), jnp.float32))
        scratch.insert(6, pltpu.VMEM_SHARED((ACC_ROWS, DEG_W), jnp.float32))
        scratch.append(pltpu.SemaphoreType.DMA)

    @functools.partial(
        pl.kernel, mesh=mesh, out_type=out_type, scratch_types=scratch,
        compiler_params=pltpu.CompilerParams(use_tc_tiling_on_sc=False),
    )
    def sc_pass(*refs):
        if with_deg:
            (src2d, dst2d, y_hbm, zeros_hbm, zeros1_hbm, ones_hbm,
             outy, outd,
             sidx, didx, bufa, bufb, onesb, accy, accd, sema, semb, sems) = refs
        else:
            (src2d, dst2d, y_hbm, zeros_hbm,
             outy,
             sidx, didx, bufa, bufb, accy, sema, semb) = refs

        cid = lax.axis_index("c")
        sid = lax.axis_index("s")
        wid = cid * NS + sid

        tile_rows = pl.ds(sid * ROWS_PER_TILE, ROWS_PER_TILE)
        pltpu.sync_copy(zeros_hbm, accy.at[tile_rows])
        if with_deg:
            pltpu.sync_copy(zeros1_hbm, accd.at[tile_rows])
            pltpu.sync_copy(ones_hbm, onesb)
        plsc.subcore_barrier()

        erow = pl.ds(wid * n_chunks, n_chunks)
        pltpu.sync_copy(src2d.at[erow], sidx)
        pltpu.sync_copy(dst2d.at[erow], didx)

        pltpu.async_copy(y_hbm.at[sidx.at[0]], bufa, sema)

        def scat(buf, j):
            if with_deg:
                c = pltpu.async_copy(buf, accy.at[didx.at[j]], sems, add=True)
                pltpu.sync_copy(onesb, accd.at[didx.at[j]], add=True)
                c.wait()
            else:
                pltpu.sync_copy(buf, accy.at[didx.at[j]], add=True)

        def pair(i, carry):
            j0 = 2 * i
            j1 = j0 + 1
            j2 = jnp.minimum(j0 + 2, n_chunks - 1)
            pltpu.async_copy(y_hbm.at[sidx.at[j1]], bufb, semb)
            pltpu.make_async_copy(y_hbm.at[sidx.at[j0]], bufa, sema).wait()
            scat(bufa, j0)
            pltpu.async_copy(y_hbm.at[sidx.at[j2]], bufa, sema)
            pltpu.make_async_copy(y_hbm.at[sidx.at[j1]], bufb, semb).wait()
            scat(bufb, j1)
            return carry

        lax.fori_loop(0, n_chunks // 2, pair, 0)
        pltpu.make_async_copy(
            y_hbm.at[sidx.at[n_chunks - 1]], bufa, sema).wait()
        plsc.subcore_barrier()

        pltpu.sync_copy(accy.at[tile_rows], outy.at[cid].at[tile_rows])
        if with_deg:
            pltpu.sync_copy(accd.at[tile_rows], outd.at[cid].at[tile_rows])

    return sc_pass


def _tc2_body(accy_ref, accd_ref, xr_ref, b1_ref, w2_ref, root2_ref,
              y2_ref, hr_ref):
    agg = accy_ref[0] + accy_ref[1]
    deg = accd_ref[0, :, 0:1] + accd_ref[1, :, 0:1]
    h = agg / jnp.maximum(deg, 1.0) + xr_ref[...] + b1_ref[...]
    h = jnp.where(h > 0.0, h, jnp.exp(jnp.minimum(h, 0.0)) - 1.0)
    wc = (1.0 - U) * w2_ref[0] + U * w2_ref[1]
    y2_ref[...] = jnp.dot(h, wc, preferred_element_type=jnp.float32)
    hr_ref[...] = jnp.dot(h, root2_ref[...], preferred_element_type=jnp.float32)


def _tc_stage2(accY, accD, XR, bias1, W2, root2):
    n = XR.shape[0]
    return pl.pallas_call(
        _tc2_body,
        grid=(n // BLK,),
        in_specs=[
            pl.BlockSpec((NC, BLK, D_HID), lambda i: (0, i, 0)),
            pl.BlockSpec((NC, BLK, DEG_W), lambda i: (0, i, 0)),
            pl.BlockSpec((BLK, D_HID), lambda i: (i, 0)),
            pl.BlockSpec((1, D_HID), lambda i: (0, 0)),
            pl.BlockSpec((2, D_HID, D_HID), lambda i: (0, 0, 0)),
            pl.BlockSpec((D_HID, D_HID), lambda i: (0, 0)),
        ],
        out_specs=[
            pl.BlockSpec((BLK, D_HID), lambda i: (i, 0)),
            pl.BlockSpec((BLK, D_HID), lambda i: (i, 0)),
        ],
        out_shape=[
            jax.ShapeDtypeStruct((n, D_HID), jnp.float32),
            jax.ShapeDtypeStruct((n, D_HID), jnp.float32),
        ],
    )(accY, accD, XR, bias1, W2, root2)


def _tc3_body(accy_ref, accd_ref, hr_ref, b2_ref, out_ref):
    agg = accy_ref[0] + accy_ref[1]
    deg = accd_ref[0, :, 0:1] + accd_ref[1, :, 0:1]
    o = agg / jnp.maximum(deg, 1.0) + hr_ref[...] + b2_ref[...]
    m = jnp.max(o, axis=1, keepdims=True)
    s = o - m
    lse = jnp.log(jnp.sum(jnp.exp(s), axis=1, keepdims=True))
    out_ref[...] = s - lse


def _tc_stage3(accY2, accD, HR, bias2):
    n = HR.shape[0]
    return pl.pallas_call(
        _tc3_body,
        grid=(n // BLK,),
        in_specs=[
            pl.BlockSpec((NC, BLK, D_HID), lambda i: (0, i, 0)),
            pl.BlockSpec((NC, BLK, DEG_W), lambda i: (0, i, 0)),
            pl.BlockSpec((BLK, D_HID), lambda i: (i, 0)),
            pl.BlockSpec((1, D_HID), lambda i: (0, 0)),
        ],
        out_specs=pl.BlockSpec((BLK, D_HID), lambda i: (i, 0)),
        out_shape=jax.ShapeDtypeStruct((n, D_HID), jnp.float32),
    )(accY2, accD, HR, bias2)


def kernel(x, edge_index, dropout, W1, root1, bias1, W2, root2, bias2):
    n = x.shape[0]
    e = edge_index.shape[1]
    n_chunks = _num_chunks(e)
    e_pad = NW * n_chunks * CHUNK

    src = edge_index[0].astype(jnp.int32)
    dst = edge_index[1].astype(jnp.int32)
    src = jnp.concatenate([src, jnp.zeros((e_pad - e,), jnp.int32)])
    dst = jnp.concatenate(
        [dst, jnp.full((e_pad - e,), DUMMY_DST, jnp.int32)])
    src2d = src.reshape(e_pad // CHUNK, CHUNK)
    dst2d = dst.reshape(e_pad // CHUNK, CHUNK)

    zeros_t = jnp.zeros((ROWS_PER_TILE, D_HID), jnp.float32)
    zeros1_t = jnp.zeros((ROWS_PER_TILE, DEG_W), jnp.float32)
    ones_t = jnp.ones((CHUNK, DEG_W), jnp.float32)

    Y1, XR = _tc_stage1(x, W1, root1)

    sc_a = _make_sc_pass(n_chunks, with_deg=True)
    accY, accD = sc_a(src2d, dst2d, Y1, zeros_t, zeros1_t, ones_t)

    Y2, HR = _tc_stage2(accY, accD, XR,
                        bias1.reshape(1, D_HID), W2, root2)

    sc_b = _make_sc_pass(n_chunks, with_deg=False)
    res = sc_b(src2d, dst2d, Y2, zeros_t)
    accY2 = res[0] if isinstance(res, (list, tuple)) else res

    return _tc_stage3(accY2, accD, HR,
                      bias2.reshape(1, D_HID))
